# baseline (speedup 1.0000x reference)
import functools

import jax
import jax.numpy as jnp
from jax import lax
from jax.experimental import pallas as pl
from jax.experimental.pallas import tpu as pltpu
from jax.experimental.pallas import tpu_sc as plsc

HEADS = 8
N = 10000
NPAD = 10112
DUMP = 10000
NSUB = 16
NCORE = 2
KE = 128


def _mlp3_body(x_ref, w1_ref, b1_ref, w2_ref, b2_ref, w3_ref, b3_ref, o_ref):
    h = jnp.maximum(x_ref[...] @ w1_ref[...] + b1_ref[...], 0.0)
    h = jnp.maximum(h @ w2_ref[...] + b2_ref[...], 0.0)
    o_ref[...] = h @ w3_ref[...] + b3_ref[...]


def _mlp3(x, p, pre, blk=1024):
    n = x.shape[0]
    ws = [p[pre + '_w1'], p[pre + '_b1'].reshape(1, -1),
          p[pre + '_w2'], p[pre + '_b2'].reshape(1, -1),
          p[pre + '_w3'], p[pre + '_b3'].reshape(1, -1)]
    grid = (pl.cdiv(n, blk),)
    out = pl.pallas_call(
        _mlp3_body,
        grid=grid,
        in_specs=[pl.BlockSpec((blk, x.shape[1]), lambda i: (i, 0))] +
                 [pl.BlockSpec(w.shape, lambda i: (0, 0)) for w in ws],
        out_specs=pl.BlockSpec((blk, ws[4].shape[1]), lambda i: (i, 0)),
        out_shape=jax.ShapeDtypeStruct((n, ws[4].shape[1]), x.dtype),
    )(x, *ws)
    return out


def _gat_agg_sc(hp_flat, asrc_p, adst_p, srcs, dsts, cp):
    nch = srcs.shape[1]
    nrows = NPAD // NSUB
    mesh = plsc.VectorSubcoreMesh(core_axis_name="c", subcore_axis_name="s")

    @functools.partial(
        pl.kernel,
        mesh=mesh,
        compiler_params=pltpu.CompilerParams(use_tc_tiling_on_sc=False,
                                             needs_layout_passes=False),
        out_type=jax.ShapeDtypeStruct((HEADS * NPAD, cp), jnp.float32),
        scratch_types=[
            pltpu.VMEM((nch, KE), jnp.int32),
            pltpu.VMEM((nch, KE), jnp.int32),
            pltpu.VMEM((NPAD,), jnp.float32),
            pltpu.VMEM((NPAD,), jnp.float32),
            pltpu.VMEM((KE,), jnp.int32),
            pltpu.VMEM((KE,), jnp.int32),
            pltpu.VMEM((KE,), jnp.float32),
            pltpu.VMEM((KE,), jnp.float32),
            pltpu.VMEM((KE, cp), jnp.float32),
            pltpu.VMEM((KE, cp), jnp.float32),
            pltpu.VMEM_SHARED((NPAD, cp), jnp.float32),
            pltpu.SemaphoreType.DMA,
            pltpu.SemaphoreType.DMA,
            pltpu.SemaphoreType.DMA,
            pltpu.SemaphoreType.DMA,
        ],
    )
    def k(hp_hbm, asrc_hbm, adst_hbm, srcs_hbm, dsts_hbm, out_hbm,
          src_t, dst_t, asrc_v, adst_v, soff0, soff1, exv0, exv1,
          rb0, rb1, accum, sg0, sg1, ss0, ss1):
        c = lax.axis_index("c")
        s = lax.axis_index("s")
        row0 = s * nrows
        pltpu.sync_copy(srcs_hbm.at[s], src_t)
        pltpu.sync_copy(dsts_hbm.at[s], dst_t)
        soff = (soff0, soff1)
        exv = (exv0, exv1)
        rb = (rb0, rb1)
        sg = (sg0, sg1)
        ss = (ss0, ss1)
        npairs = nch // 2

        for g in range(HEADS // NCORE):
            h = c * (HEADS // NCORE) + g
            pltpu.sync_copy(asrc_hbm.at[h], asrc_v)
            pltpu.sync_copy(adst_hbm.at[h], adst_v)
            hoff = h * NPAD

            def zrow(i, _):
                for f in range(cp // 16):
                    rb0[i, pl.ds(f * 16, 16)] = jnp.zeros((16,), jnp.float32)
                return 0
            lax.fori_loop(0, KE, zrow, 0)
            full = nrows // KE
            for z in range(full):
                pltpu.sync_copy(rb0, accum.at[pl.ds(row0 + z * KE, KE)])
            rem = nrows - full * KE
            if rem:
                pltpu.sync_copy(rb0.at[pl.ds(0, rem)],
                                accum.at[pl.ds(row0 + full * KE, rem)])
            plsc.subcore_barrier()

            def prep(j, b):
                def exstep(v, _):
                    s16 = src_t[j, pl.ds(v * 16, 16)]
                    d16 = dst_t[j, pl.ds(v * 16, 16)]
                    av = plsc.load_gather(asrc_v, [s16])
                    bv = plsc.load_gather(adst_v, [d16])
                    e = av + bv
                    e = jnp.where(e >= 0.0, e, 0.2 * e)
                    exv[b][pl.ds(v * 16, 16)] = jnp.exp(e)
                    soff[b][pl.ds(v * 16, 16)] = s16 + hoff
                    return 0
                lax.fori_loop(0, KE // 16, exstep, 0)

            def gstart(b):
                pltpu.async_copy(hp_hbm.at[soff[b]], rb[b], sg[b])

            def gwait(b):
                pltpu.make_async_copy(hp_hbm.at[soff[b]], rb[b], sg[b]).wait()

            def scale(b):
                def scstep(i, _):
                    m = plsc.load_gather(
                        exv[b], [jnp.full((16,), 1, jnp.int32) * i])
                    for f in range(cp // 16):
                        rb[b][i, pl.ds(f * 16, 16)] = (
                            rb[b][i, pl.ds(f * 16, 16)] * m)
                    return 0
                lax.fori_loop(0, KE, scstep, 0)

            def scstart(j, b):
                pltpu.async_copy(rb[b], accum.at[dst_t.at[j]], ss[b], add=True)

            def scwait(j, b):
                pltpu.make_async_copy(
                    rb[b], accum.at[dst_t.at[j]], ss[b]).wait()

            prep(0, 0)
            gstart(0)

            def pair_body(jp, _):
                j0 = 2 * jp
                prep(j0 + 1, 1)

                @pl.when(jp >= 1)
                def _():
                    scwait(j0 - 1, 1)
                gstart(1)
                gwait(0)
                scale(0)
                scstart(j0, 0)
                @pl.when(jp + 1 < npairs)
                def _():
                    prep(j0 + 2, 0)
                    scwait(j0, 0)
                    gstart(0)
                gwait(1)
                scale(1)
                scstart(j0 + 1, 1)
                return 0

            lax.fori_loop(0, npairs, pair_body, 0)
            scwait(nch - 2, 0)
            scwait(nch - 1, 1)
            plsc.subcore_barrier()

            pltpu.sync_copy(accum.at[pl.ds(row0, nrows)],
                            out_hbm.at[pl.ds(hoff + row0, nrows)])
            plsc.subcore_barrier()

    return k(hp_flat, asrc_p, adst_p, srcs, dsts)


def _gat(x, srcs, dsts, p, pre, out_ch):
    c = out_ch
    h = (x @ p[pre + '_w']).reshape(N, HEADS, c)
    a_src = jnp.sum(h * p[pre + '_asrc'][None], axis=-1)
    a_dst = jnp.sum(h * p[pre + '_adst'][None], axis=-1)
    asrc_p = jnp.zeros((HEADS, NPAD), jnp.float32).at[:, :N].set(a_src.T)
    adst_p = jnp.zeros((HEADS, NPAD), jnp.float32).at[:, :N].set(a_dst.T)

    ht = jnp.transpose(h, (1, 0, 2))
    parts = []
    for f0 in range(0, c, 64):
        fw = min(64, c - f0)
        cpp = fw + 16
        hp = jnp.zeros((HEADS, NPAD, cpp), jnp.float32)
        hp = hp.at[:, :N, :fw].set(ht[:, :, f0:f0 + fw])
        hp = hp.at[:, :N, fw].set(1.0)
        agg = _gat_agg_sc(hp.reshape(HEADS * NPAD, cpp), asrc_p, adst_p,
                          srcs, dsts, cpp)
        agg = agg.reshape(HEADS, NPAD, cpp)
        parts.append(agg[:, :N, :fw] / (agg[:, :N, fw:fw + 1] + 1e-16))
    outg = jnp.concatenate(parts, axis=-1) if len(parts) > 1 else parts[0]
    out = jnp.transpose(outg, (1, 0, 2)).reshape(N, HEADS * c)
    return out + p[pre + '_b']


def _knn_body(k, nn, hb_ref, hall_ref, o_ref, d_ref):
    hb = hb_ref[...]
    ha = hall_ref[...]
    r = hb.shape[0]
    x2b = jnp.sum(hb * hb, axis=1, keepdims=True)
    x2a = jnp.sum(ha * ha, axis=1)[None, :]
    dot = lax.dot_general(hb, ha, (((1,), (1,)), ((), ())),
                          preferred_element_type=jnp.float32)
    neg = 2.0 * dot - x2b - x2a
    d_ref[...] = neg
    m0 = jnp.max(neg, axis=1, keepdims=True)
    iota = lax.broadcasted_iota(jnp.int32, (r, nn), 1)
    kiota = lax.broadcasted_iota(jnp.int32, (r, k), 1)

    def step(kk, carry):
        idxmat, m = carry
        cand = d_ref[...]
        idxv = jnp.min(jnp.where(cand == m, iota, nn), axis=1, keepdims=True)
        cand = jnp.where(iota == idxv, -3.4e38, cand)
        d_ref[...] = cand
        m2 = jnp.max(cand, axis=1, keepdims=True)
        return jnp.where(kiota == kk, idxv, idxmat), m2

    o_ref[...] = lax.fori_loop(
        0, k, step, (jnp.zeros((r, k), jnp.int32), m0))[0]


def _knn_idx(x, k, blk=256):
    n, c = x.shape
    cpad = (-c) % 128
    if cpad:
        x = jnp.pad(x, ((0, 0), (0, cpad)))
        c += cpad
    grid = (pl.cdiv(n, blk),)
    idx = pl.pallas_call(
        functools.partial(_knn_body, k, n),
        grid=grid,
        in_specs=[pl.BlockSpec((blk, c), lambda i: (i, 0)),
                  pl.BlockSpec((n, c), lambda i: (0, 0))],
        out_specs=pl.BlockSpec((blk, k), lambda i: (i, 0)),
        out_shape=jax.ShapeDtypeStruct((n, k), jnp.int32),
        scratch_shapes=[pltpu.VMEM((blk, n), jnp.float32)],
        compiler_params=pltpu.CompilerParams(
            vmem_limit_bytes=100 * 1024 * 1024),
    )(x, x)
    return idx


def _edge_conv(h, k, p, pre):
    idx = _knn_idx(h, k)
    idx = lax.stop_gradient(idx)
    hj = h[idx]
    hi = jnp.broadcast_to(h[:, None, :], hj.shape)
    m = jnp.concatenate([hi, hj - hi], axis=-1)
    m = jax.nn.relu(m @ p[pre + '_w1'] + p[pre + '_b1'])
    m = m @ p[pre + '_w2'] + p[pre + '_b2']
    return jnp.max(m, axis=1)


def kernel(x, pos, batch, edge_index, params):
    p = params
    loops = jnp.arange(N, dtype=edge_index.dtype)
    src = jnp.concatenate([edge_index[0], loops])
    dst = jnp.concatenate([edge_index[1], loops])
    e2 = src.shape[0]
    e2p = ((e2 + 2 * NSUB * KE - 1) // (2 * NSUB * KE)) * (2 * NSUB * KE)
    src_p = jnp.concatenate([src, jnp.zeros((e2p - e2,), jnp.int32)])
    dst_p = jnp.concatenate([dst, jnp.full((e2p - e2,), DUMP, jnp.int32)])
    srcs = src_p.reshape(NSUB, -1, KE)
    dsts = dst_p.reshape(NSUB, -1, KE)

    x_surf = x[:, :39]
    xp = _mlp3(x[:, 39:1063], p, 'progen2')
    xd = _mlp3(x[:, 1063:2087], p, 'distarr')
    x0 = jnp.concatenate([x_surf, xp, xd], axis=1)
    x1 = _edge_conv(x0, 20, p, 'conv1')
    x2 = _edge_conv(x1, 20, p, 'conv2')
    x3 = _edge_conv(x2, 20, p, 'conv3')
    x3 = jnp.concatenate([x3, x_surf, xp, xd], axis=1)
    x4 = jax.nn.elu(_gat(x3, srcs, dsts, p, 'gat1', 128))
    x5 = jax.nn.elu(_gat(x4, srcs, dsts, p, 'gat2', 64))
    x6 = jax.nn.elu(_gat(x5, srcs, dsts, p, 'gat3', 32))
    x6 = jnp.concatenate([x6, x3], axis=1)
    out = _mlp3(x6, p, 'head')
    return jax.nn.sigmoid(5.0 * out)


# speedup vs baseline: 1.0171x; 1.0171x over previous
import functools

import jax
import jax.numpy as jnp
from jax import lax
from jax.experimental import pallas as pl
from jax.experimental.pallas import tpu as pltpu
from jax.experimental.pallas import tpu_sc as plsc

HEADS = 8
N = 10000
NPAD = 10112
DUMP = 10000
NSUB = 16
NCORE = 2
KE = 128


def _mlp3_body(x_ref, w1_ref, b1_ref, w2_ref, b2_ref, w3_ref, b3_ref, o_ref):
    h = jnp.maximum(x_ref[...] @ w1_ref[...] + b1_ref[...], 0.0)
    h = jnp.maximum(h @ w2_ref[...] + b2_ref[...], 0.0)
    o_ref[...] = h @ w3_ref[...] + b3_ref[...]


def _mlp3(x, p, pre, blk=1024):
    n = x.shape[0]
    ws = [p[pre + '_w1'], p[pre + '_b1'].reshape(1, -1),
          p[pre + '_w2'], p[pre + '_b2'].reshape(1, -1),
          p[pre + '_w3'], p[pre + '_b3'].reshape(1, -1)]
    grid = (pl.cdiv(n, blk),)
    out = pl.pallas_call(
        _mlp3_body,
        grid=grid,
        in_specs=[pl.BlockSpec((blk, x.shape[1]), lambda i: (i, 0))] +
                 [pl.BlockSpec(w.shape, lambda i: (0, 0)) for w in ws],
        out_specs=pl.BlockSpec((blk, ws[4].shape[1]), lambda i: (i, 0)),
        out_shape=jax.ShapeDtypeStruct((n, ws[4].shape[1]), x.dtype),
    )(x, *ws)
    return out


def _gat_agg_sc(hp_flat, asrc_p, adst_p, srcs, dsts, cp):
    nch = srcs.shape[1]
    nrows = NPAD // NSUB
    mesh = plsc.VectorSubcoreMesh(core_axis_name="c", subcore_axis_name="s")

    @functools.partial(
        pl.kernel,
        mesh=mesh,
        compiler_params=pltpu.CompilerParams(use_tc_tiling_on_sc=False,
                                             needs_layout_passes=False),
        out_type=jax.ShapeDtypeStruct((HEADS * NPAD, cp), jnp.float32),
        scratch_types=[
            pltpu.VMEM((nch, KE), jnp.int32),
            pltpu.VMEM((nch, KE), jnp.int32),
            pltpu.VMEM((NPAD,), jnp.float32),
            pltpu.VMEM((NPAD,), jnp.float32),
            pltpu.VMEM((KE,), jnp.int32),
            pltpu.VMEM((KE,), jnp.int32),
            pltpu.VMEM((KE,), jnp.float32),
            pltpu.VMEM((KE,), jnp.float32),
            pltpu.VMEM((KE, cp), jnp.float32),
            pltpu.VMEM((KE, cp), jnp.float32),
            pltpu.VMEM_SHARED((NPAD, cp), jnp.float32),
            pltpu.SemaphoreType.DMA,
            pltpu.SemaphoreType.DMA,
            pltpu.SemaphoreType.DMA,
            pltpu.SemaphoreType.DMA,
        ],
    )
    def k(hp_hbm, asrc_hbm, adst_hbm, srcs_hbm, dsts_hbm, out_hbm,
          src_t, dst_t, asrc_v, adst_v, soff0, soff1, exv0, exv1,
          rb0, rb1, accum, sg0, sg1, ss0, ss1):
        c = lax.axis_index("c")
        s = lax.axis_index("s")
        row0 = s * nrows
        pltpu.sync_copy(srcs_hbm.at[s], src_t)
        pltpu.sync_copy(dsts_hbm.at[s], dst_t)
        soff = (soff0, soff1)
        exv = (exv0, exv1)
        rb = (rb0, rb1)
        sg = (sg0, sg1)
        ss = (ss0, ss1)
        npairs = nch // 2

        for g in range(HEADS // NCORE):
            h = c * (HEADS // NCORE) + g
            pltpu.sync_copy(asrc_hbm.at[h], asrc_v)
            pltpu.sync_copy(adst_hbm.at[h], adst_v)
            hoff = h * NPAD

            def zrow(i, _):
                for f in range(cp // 16):
                    rb0[i, pl.ds(f * 16, 16)] = jnp.zeros((16,), jnp.float32)
                return 0
            lax.fori_loop(0, KE, zrow, 0)
            full = nrows // KE
            for z in range(full):
                pltpu.sync_copy(rb0, accum.at[pl.ds(row0 + z * KE, KE)])
            rem = nrows - full * KE
            if rem:
                pltpu.sync_copy(rb0.at[pl.ds(0, rem)],
                                accum.at[pl.ds(row0 + full * KE, rem)])
            plsc.subcore_barrier()

            def prep(j, b):
                def exstep(v, _):
                    s16 = src_t[j, pl.ds(v * 16, 16)]
                    d16 = dst_t[j, pl.ds(v * 16, 16)]
                    av = plsc.load_gather(asrc_v, [s16])
                    bv = plsc.load_gather(adst_v, [d16])
                    e = av + bv
                    e = jnp.where(e >= 0.0, e, 0.2 * e)
                    exv[b][pl.ds(v * 16, 16)] = jnp.exp(e)
                    soff[b][pl.ds(v * 16, 16)] = s16 + hoff
                    return 0
                lax.fori_loop(0, KE // 16, exstep, 0)

            def gstart(b):
                pltpu.async_copy(hp_hbm.at[soff[b]], rb[b], sg[b])

            def gwait(b):
                pltpu.make_async_copy(hp_hbm.at[soff[b]], rb[b], sg[b]).wait()

            def scale(b):
                def scstep(i, _):
                    m = plsc.load_gather(
                        exv[b], [jnp.full((16,), 1, jnp.int32) * i])
                    for f in range(cp // 16):
                        rb[b][i, pl.ds(f * 16, 16)] = (
                            rb[b][i, pl.ds(f * 16, 16)] * m)
                    return 0
                lax.fori_loop(0, KE, scstep, 0)

            def scstart(j, b):
                pltpu.async_copy(rb[b], accum.at[dst_t.at[j]], ss[b], add=True)

            def scwait(j, b):
                pltpu.make_async_copy(
                    rb[b], accum.at[dst_t.at[j]], ss[b]).wait()

            prep(0, 0)
            gstart(0)

            def pair_body(jp, _):
                j0 = 2 * jp
                prep(j0 + 1, 1)

                @pl.when(jp >= 1)
                def _():
                    scwait(j0 - 1, 1)
                gstart(1)
                gwait(0)
                scale(0)
                scstart(j0, 0)
                @pl.when(jp + 1 < npairs)
                def _():
                    prep(j0 + 2, 0)
                    scwait(j0, 0)
                    gstart(0)
                gwait(1)
                scale(1)
                scstart(j0 + 1, 1)
                return 0

            lax.fori_loop(0, npairs, pair_body, 0)
            scwait(nch - 2, 0)
            scwait(nch - 1, 1)
            plsc.subcore_barrier()

            pltpu.sync_copy(accum.at[pl.ds(row0, nrows)],
                            out_hbm.at[pl.ds(hoff + row0, nrows)])
            plsc.subcore_barrier()

    return k(hp_flat, asrc_p, adst_p, srcs, dsts)


def _gat(x, srcs, dsts, p, pre, out_ch):
    c = out_ch
    h = (x @ p[pre + '_w']).reshape(N, HEADS, c)
    a_src = jnp.sum(h * p[pre + '_asrc'][None], axis=-1)
    a_dst = jnp.sum(h * p[pre + '_adst'][None], axis=-1)
    asrc_p = jnp.zeros((HEADS, NPAD), jnp.float32).at[:, :N].set(a_src.T)
    adst_p = jnp.zeros((HEADS, NPAD), jnp.float32).at[:, :N].set(a_dst.T)

    ht = jnp.transpose(h, (1, 0, 2))
    parts = []
    for f0 in range(0, c, 64):
        fw = min(64, c - f0)
        cpp = fw + 16
        hp = jnp.zeros((HEADS, NPAD, cpp), jnp.float32)
        hp = hp.at[:, :N, :fw].set(ht[:, :, f0:f0 + fw])
        hp = hp.at[:, :N, fw].set(1.0)
        agg = _gat_agg_sc(hp.reshape(HEADS * NPAD, cpp), asrc_p, adst_p,
                          srcs, dsts, cpp)
        agg = agg.reshape(HEADS, NPAD, cpp)
        parts.append(agg[:, :N, :fw] / (agg[:, :N, fw:fw + 1] + 1e-16))
    outg = jnp.concatenate(parts, axis=-1) if len(parts) > 1 else parts[0]
    out = jnp.transpose(outg, (1, 0, 2)).reshape(N, HEADS * c)
    return out + p[pre + '_b']


def _knn_body(k, nn, hb_ref, hall_ref, o_ref, d_ref):
    hb = hb_ref[...]
    ha = hall_ref[...]
    r = hb.shape[0]
    x2b = jnp.sum(hb * hb, axis=1, keepdims=True)
    x2a = jnp.sum(ha * ha, axis=1)[None, :]
    dot = lax.dot_general(hb, ha, (((1,), (1,)), ((), ())),
                          preferred_element_type=jnp.float32)
    d_ref[...] = 2.0 * dot - x2b - x2a
    iota = lax.broadcasted_iota(jnp.int32, (r, nn), 1)
    kiota = lax.broadcasted_iota(jnp.int32, (r, k), 1)

    def step(kk, idxmat):
        cand = d_ref[...]
        m = jnp.max(cand, axis=1, keepdims=True)
        idxv = jnp.min(jnp.where(cand == m, iota, nn), axis=1, keepdims=True)
        d_ref[...] = jnp.where(iota == idxv, -3.4e38, cand)
        return jnp.where(kiota == kk, idxv, idxmat)

    o_ref[...] = lax.fori_loop(0, k, step, jnp.zeros((r, k), jnp.int32))


def _knn_idx(x, k, blk=256):
    n, c = x.shape
    cpad = (-c) % 128
    if cpad:
        x = jnp.pad(x, ((0, 0), (0, cpad)))
        c += cpad
    grid = (pl.cdiv(n, blk),)
    idx = pl.pallas_call(
        functools.partial(_knn_body, k, n),
        grid=grid,
        in_specs=[pl.BlockSpec((blk, c), lambda i: (i, 0)),
                  pl.BlockSpec((n, c), lambda i: (0, 0))],
        out_specs=pl.BlockSpec((blk, k), lambda i: (i, 0)),
        out_shape=jax.ShapeDtypeStruct((n, k), jnp.int32),
        scratch_shapes=[pltpu.VMEM((blk, n), jnp.float32)],
        compiler_params=pltpu.CompilerParams(
            vmem_limit_bytes=100 * 1024 * 1024),
    )(x, x)
    return idx


def _edge_conv(h, k, p, pre):
    idx = _knn_idx(h, k)
    idx = lax.stop_gradient(idx)
    hj = h[idx]
    hi = jnp.broadcast_to(h[:, None, :], hj.shape)
    m = jnp.concatenate([hi, hj - hi], axis=-1)
    m = jax.nn.relu(m @ p[pre + '_w1'] + p[pre + '_b1'])
    m = m @ p[pre + '_w2'] + p[pre + '_b2']
    return jnp.max(m, axis=1)


def kernel(x, pos, batch, edge_index, params):
    p = params
    loops = jnp.arange(N, dtype=edge_index.dtype)
    src = jnp.concatenate([edge_index[0], loops])
    dst = jnp.concatenate([edge_index[1], loops])
    e2 = src.shape[0]
    e2p = ((e2 + 2 * NSUB * KE - 1) // (2 * NSUB * KE)) * (2 * NSUB * KE)
    src_p = jnp.concatenate([src, jnp.zeros((e2p - e2,), jnp.int32)])
    dst_p = jnp.concatenate([dst, jnp.full((e2p - e2,), DUMP, jnp.int32)])
    srcs = src_p.reshape(NSUB, -1, KE)
    dsts = dst_p.reshape(NSUB, -1, KE)

    x_surf = x[:, :39]
    xp = _mlp3(x[:, 39:1063], p, 'progen2')
    xd = _mlp3(x[:, 1063:2087], p, 'distarr')
    x0 = jnp.concatenate([x_surf, xp, xd], axis=1)
    x1 = _edge_conv(x0, 20, p, 'conv1')
    x2 = _edge_conv(x1, 20, p, 'conv2')
    x3 = _edge_conv(x2, 20, p, 'conv3')
    x3 = jnp.concatenate([x3, x_surf, xp, xd], axis=1)
    x4 = jax.nn.elu(_gat(x3, srcs, dsts, p, 'gat1', 128))
    x5 = jax.nn.elu(_gat(x4, srcs, dsts, p, 'gat2', 64))
    x6 = jax.nn.elu(_gat(x5, srcs, dsts, p, 'gat3', 32))
    x6 = jnp.concatenate([x6, x3], axis=1)
    out = _mlp3(x6, p, 'head')
    return jax.nn.sigmoid(5.0 * out)


# speedup vs baseline: 1.0260x; 1.0087x over previous
import functools

import jax
import jax.numpy as jnp
from jax import lax
from jax.experimental import pallas as pl
from jax.experimental.pallas import tpu as pltpu
from jax.experimental.pallas import tpu_sc as plsc

HEADS = 8
N = 10000
NPAD = 10112
DUMP = 10000
NSUB = 16
NCORE = 2
KE = 128


def _mlp3_body(x_ref, w1_ref, b1_ref, w2_ref, b2_ref, w3_ref, b3_ref, o_ref):
    h = jnp.maximum(x_ref[...] @ w1_ref[...] + b1_ref[...], 0.0)
    h = jnp.maximum(h @ w2_ref[...] + b2_ref[...], 0.0)
    o_ref[...] = h @ w3_ref[...] + b3_ref[...]


def _mlp3(x, p, pre, blk=1024):
    n = x.shape[0]
    ws = [p[pre + '_w1'], p[pre + '_b1'].reshape(1, -1),
          p[pre + '_w2'], p[pre + '_b2'].reshape(1, -1),
          p[pre + '_w3'], p[pre + '_b3'].reshape(1, -1)]
    grid = (pl.cdiv(n, blk),)
    out = pl.pallas_call(
        _mlp3_body,
        grid=grid,
        in_specs=[pl.BlockSpec((blk, x.shape[1]), lambda i: (i, 0))] +
                 [pl.BlockSpec(w.shape, lambda i: (0, 0)) for w in ws],
        out_specs=pl.BlockSpec((blk, ws[4].shape[1]), lambda i: (i, 0)),
        out_shape=jax.ShapeDtypeStruct((n, ws[4].shape[1]), x.dtype),
    )(x, *ws)
    return out


def _gat_agg_sc(hp_flat, asrc_p, adst_p, srcs, dsts, cp):
    nch = srcs.shape[1]
    nrows = NPAD // NSUB
    mesh = plsc.VectorSubcoreMesh(core_axis_name="c", subcore_axis_name="s")

    @functools.partial(
        pl.kernel,
        mesh=mesh,
        compiler_params=pltpu.CompilerParams(use_tc_tiling_on_sc=False,
                                             needs_layout_passes=False),
        out_type=jax.ShapeDtypeStruct((HEADS * NPAD, cp), jnp.float32),
        scratch_types=[
            pltpu.VMEM((nch, KE), jnp.int32),
            pltpu.VMEM((nch, KE), jnp.int32),
            pltpu.VMEM((NPAD,), jnp.float32),
            pltpu.VMEM((NPAD,), jnp.float32),
            pltpu.VMEM((KE,), jnp.int32),
            pltpu.VMEM((KE,), jnp.int32),
            pltpu.VMEM((KE,), jnp.float32),
            pltpu.VMEM((KE,), jnp.float32),
            pltpu.VMEM((KE, cp), jnp.float32),
            pltpu.VMEM((KE, cp), jnp.float32),
            pltpu.VMEM_SHARED((NPAD, cp), jnp.float32),
            pltpu.SemaphoreType.DMA,
            pltpu.SemaphoreType.DMA,
            pltpu.SemaphoreType.DMA,
            pltpu.SemaphoreType.DMA,
        ],
    )
    def k(hp_hbm, asrc_hbm, adst_hbm, srcs_hbm, dsts_hbm, out_hbm,
          src_t, dst_t, asrc_v, adst_v, soff0, soff1, exv0, exv1,
          rb0, rb1, accum, sg0, sg1, ss0, ss1):
        c = lax.axis_index("c")
        s = lax.axis_index("s")
        row0 = s * nrows
        pltpu.sync_copy(srcs_hbm.at[s], src_t)
        pltpu.sync_copy(dsts_hbm.at[s], dst_t)
        soff = (soff0, soff1)
        exv = (exv0, exv1)
        rb = (rb0, rb1)
        sg = (sg0, sg1)
        ss = (ss0, ss1)
        npairs = nch // 2

        for g in range(HEADS // NCORE):
            h = c * (HEADS // NCORE) + g
            pltpu.sync_copy(asrc_hbm.at[h], asrc_v)
            pltpu.sync_copy(adst_hbm.at[h], adst_v)
            hoff = h * NPAD

            def zrow(i, _):
                for f in range(cp // 16):
                    rb0[i, pl.ds(f * 16, 16)] = jnp.zeros((16,), jnp.float32)
                return 0
            lax.fori_loop(0, KE, zrow, 0)
            full = nrows // KE
            for z in range(full):
                pltpu.sync_copy(rb0, accum.at[pl.ds(row0 + z * KE, KE)])
            rem = nrows - full * KE
            if rem:
                pltpu.sync_copy(rb0.at[pl.ds(0, rem)],
                                accum.at[pl.ds(row0 + full * KE, rem)])
            plsc.subcore_barrier()

            def prep(j, b):
                def exstep(v, _):
                    s16 = src_t[j, pl.ds(v * 16, 16)]
                    d16 = dst_t[j, pl.ds(v * 16, 16)]
                    av = plsc.load_gather(asrc_v, [s16])
                    bv = plsc.load_gather(adst_v, [d16])
                    e = av + bv
                    e = jnp.where(e >= 0.0, e, 0.2 * e)
                    exv[b][pl.ds(v * 16, 16)] = jnp.exp(e)
                    soff[b][pl.ds(v * 16, 16)] = s16 + hoff
                    return 0
                lax.fori_loop(0, KE // 16, exstep, 0)

            def gstart(b):
                pltpu.async_copy(hp_hbm.at[soff[b]], rb[b], sg[b])

            def gwait(b):
                pltpu.make_async_copy(hp_hbm.at[soff[b]], rb[b], sg[b]).wait()

            def scale(b):
                def scstep(i, _):
                    m = plsc.load_gather(
                        exv[b], [jnp.full((16,), 1, jnp.int32) * i])
                    for f in range(cp // 16):
                        rb[b][i, pl.ds(f * 16, 16)] = (
                            rb[b][i, pl.ds(f * 16, 16)] * m)
                    return 0
                lax.fori_loop(0, KE, scstep, 0)

            def scstart(j, b):
                pltpu.async_copy(rb[b], accum.at[dst_t.at[j]], ss[b], add=True)

            def scwait(j, b):
                pltpu.make_async_copy(
                    rb[b], accum.at[dst_t.at[j]], ss[b]).wait()

            prep(0, 0)
            gstart(0)

            def pair_body(jp, _):
                j0 = 2 * jp
                prep(j0 + 1, 1)

                @pl.when(jp >= 1)
                def _():
                    scwait(j0 - 1, 1)
                gstart(1)
                gwait(0)
                scale(0)
                scstart(j0, 0)
                @pl.when(jp + 1 < npairs)
                def _():
                    prep(j0 + 2, 0)
                    scwait(j0, 0)
                    gstart(0)
                gwait(1)
                scale(1)
                scstart(j0 + 1, 1)
                return 0

            lax.fori_loop(0, npairs, pair_body, 0)
            scwait(nch - 2, 0)
            scwait(nch - 1, 1)
            plsc.subcore_barrier()

            pltpu.sync_copy(accum.at[pl.ds(row0, nrows)],
                            out_hbm.at[pl.ds(hoff + row0, nrows)])
            plsc.subcore_barrier()

    return k(hp_flat, asrc_p, adst_p, srcs, dsts)


def _gat(x, srcs, dsts, p, pre, out_ch):
    c = out_ch
    h = (x @ p[pre + '_w']).reshape(N, HEADS, c)
    a_src = jnp.sum(h * p[pre + '_asrc'][None], axis=-1)
    a_dst = jnp.sum(h * p[pre + '_adst'][None], axis=-1)
    asrc_p = jnp.zeros((HEADS, NPAD), jnp.float32).at[:, :N].set(a_src.T)
    adst_p = jnp.zeros((HEADS, NPAD), jnp.float32).at[:, :N].set(a_dst.T)

    ht = jnp.transpose(h, (1, 0, 2))
    parts = []
    for f0 in range(0, c, 64):
        fw = min(64, c - f0)
        cpp = fw + 16
        hp = jnp.zeros((HEADS, NPAD, cpp), jnp.float32)
        hp = hp.at[:, :N, :fw].set(ht[:, :, f0:f0 + fw])
        hp = hp.at[:, :N, fw].set(1.0)
        agg = _gat_agg_sc(hp.reshape(HEADS * NPAD, cpp), asrc_p, adst_p,
                          srcs, dsts, cpp)
        agg = agg.reshape(HEADS, NPAD, cpp)
        parts.append(agg[:, :N, :fw] / (agg[:, :N, fw:fw + 1] + 1e-16))
    outg = jnp.concatenate(parts, axis=-1) if len(parts) > 1 else parts[0]
    out = jnp.transpose(outg, (1, 0, 2)).reshape(N, HEADS * c)
    return out + p[pre + '_b']


def _knn_body(k, nn, hb_ref, hall_ref, o_ref, d_ref):
    hb = hb_ref[...]
    ha = hall_ref[...]
    r = hb.shape[0]
    x2b = jnp.sum(hb * hb, axis=1, keepdims=True)
    x2a = jnp.sum(ha * ha, axis=1)[None, :]
    dot = lax.dot_general(hb, ha, (((1,), (1,)), ((), ())),
                          preferred_element_type=jnp.float32)
    d_ref[...] = 2.0 * dot - x2b - x2a
    iota = lax.broadcasted_iota(jnp.int32, (r, nn), 1)
    kiota = lax.broadcasted_iota(jnp.int32, (r, k), 1)

    def step(kk, idxmat):
        cand = d_ref[...]
        m = jnp.max(cand, axis=1, keepdims=True)
        idxv = jnp.min(jnp.where(cand == m, iota, nn), axis=1, keepdims=True)
        d_ref[...] = jnp.where(iota == idxv, -3.4e38, cand)
        return jnp.where(kiota == kk, idxv, idxmat)

    o_ref[...] = lax.fori_loop(0, k, step, jnp.zeros((r, k), jnp.int32))


def _knn_idx(x, k, blk=512):
    n, c = x.shape
    cpad = (-c) % 128
    if cpad:
        x = jnp.pad(x, ((0, 0), (0, cpad)))
        c += cpad
    npad = (-n) % 512
    nn = n + npad
    if npad:
        x = jnp.pad(x, ((0, npad), (0, 0)), constant_values=1e6)
    grid = (nn // blk,)
    idx = pl.pallas_call(
        functools.partial(_knn_body, k, nn),
        grid=grid,
        in_specs=[pl.BlockSpec((blk, c), lambda i: (i, 0)),
                  pl.BlockSpec((nn, c), lambda i: (0, 0))],
        out_specs=pl.BlockSpec((blk, k), lambda i: (i, 0)),
        out_shape=jax.ShapeDtypeStruct((nn, k), jnp.int32),
        scratch_shapes=[pltpu.VMEM((blk, nn), jnp.float32)],
        compiler_params=pltpu.CompilerParams(
            vmem_limit_bytes=100 * 1024 * 1024),
    )(x, x)
    return idx[:n]


def _edge_conv(h, k, p, pre):
    idx = _knn_idx(h, k)
    idx = lax.stop_gradient(idx)
    hj = h[idx]
    hi = jnp.broadcast_to(h[:, None, :], hj.shape)
    m = jnp.concatenate([hi, hj - hi], axis=-1)
    m = jax.nn.relu(m @ p[pre + '_w1'] + p[pre + '_b1'])
    m = m @ p[pre + '_w2'] + p[pre + '_b2']
    return jnp.max(m, axis=1)


def kernel(x, pos, batch, edge_index, params):
    p = params
    loops = jnp.arange(N, dtype=edge_index.dtype)
    src = jnp.concatenate([edge_index[0], loops])
    dst = jnp.concatenate([edge_index[1], loops])
    e2 = src.shape[0]
    e2p = ((e2 + 2 * NSUB * KE - 1) // (2 * NSUB * KE)) * (2 * NSUB * KE)
    src_p = jnp.concatenate([src, jnp.zeros((e2p - e2,), jnp.int32)])
    dst_p = jnp.concatenate([dst, jnp.full((e2p - e2,), DUMP, jnp.int32)])
    srcs = src_p.reshape(NSUB, -1, KE)
    dsts = dst_p.reshape(NSUB, -1, KE)

    x_surf = x[:, :39]
    xp = _mlp3(x[:, 39:1063], p, 'progen2')
    xd = _mlp3(x[:, 1063:2087], p, 'distarr')
    x0 = jnp.concatenate([x_surf, xp, xd], axis=1)
    x1 = _edge_conv(x0, 20, p, 'conv1')
    x2 = _edge_conv(x1, 20, p, 'conv2')
    x3 = _edge_conv(x2, 20, p, 'conv3')
    x3 = jnp.concatenate([x3, x_surf, xp, xd], axis=1)
    x4 = jax.nn.elu(_gat(x3, srcs, dsts, p, 'gat1', 128))
    x5 = jax.nn.elu(_gat(x4, srcs, dsts, p, 'gat2', 64))
    x6 = jax.nn.elu(_gat(x5, srcs, dsts, p, 'gat3', 32))
    x6 = jnp.concatenate([x6, x3], axis=1)
    out = _mlp3(x6, p, 'head')
    return jax.nn.sigmoid(5.0 * out)


# speedup vs baseline: 1.0980x; 1.0702x over previous
import functools

import jax
import jax.numpy as jnp
from jax import lax
from jax.experimental import pallas as pl
from jax.experimental.pallas import tpu as pltpu
from jax.experimental.pallas import tpu_sc as plsc

HEADS = 8
N = 10000
NPAD = 10112
DUMP = 10000
NSUB = 16
NCORE = 2
KE = 128


def _mlp3_body(x_ref, w1_ref, b1_ref, w2_ref, b2_ref, w3_ref, b3_ref, o_ref):
    h = jnp.maximum(x_ref[...] @ w1_ref[...] + b1_ref[...], 0.0)
    h = jnp.maximum(h @ w2_ref[...] + b2_ref[...], 0.0)
    o_ref[...] = h @ w3_ref[...] + b3_ref[...]


def _mlp3(x, p, pre, blk=1024):
    n = x.shape[0]
    ws = [p[pre + '_w1'], p[pre + '_b1'].reshape(1, -1),
          p[pre + '_w2'], p[pre + '_b2'].reshape(1, -1),
          p[pre + '_w3'], p[pre + '_b3'].reshape(1, -1)]
    grid = (pl.cdiv(n, blk),)
    out = pl.pallas_call(
        _mlp3_body,
        grid=grid,
        in_specs=[pl.BlockSpec((blk, x.shape[1]), lambda i: (i, 0))] +
                 [pl.BlockSpec(w.shape, lambda i: (0, 0)) for w in ws],
        out_specs=pl.BlockSpec((blk, ws[4].shape[1]), lambda i: (i, 0)),
        out_shape=jax.ShapeDtypeStruct((n, ws[4].shape[1]), x.dtype),
    )(x, *ws)
    return out


def _gat_agg_sc(hp_flat, asrc_p, adst_p, srcs, dsts, cp):
    nch = srcs.shape[1]
    nrows = NPAD // NSUB
    mesh = plsc.VectorSubcoreMesh(core_axis_name="c", subcore_axis_name="s")

    @functools.partial(
        pl.kernel,
        mesh=mesh,
        compiler_params=pltpu.CompilerParams(use_tc_tiling_on_sc=False,
                                             needs_layout_passes=False),
        out_type=jax.ShapeDtypeStruct((HEADS * NPAD, cp), jnp.float32),
        scratch_types=[
            pltpu.VMEM((nch, KE), jnp.int32),
            pltpu.VMEM((nch, KE), jnp.int32),
            pltpu.VMEM((NPAD,), jnp.float32),
            pltpu.VMEM((NPAD,), jnp.float32),
            pltpu.VMEM((KE,), jnp.int32),
            pltpu.VMEM((KE,), jnp.int32),
            pltpu.VMEM((KE,), jnp.float32),
            pltpu.VMEM((KE,), jnp.float32),
            pltpu.VMEM((KE, cp), jnp.float32),
            pltpu.VMEM((KE, cp), jnp.float32),
            pltpu.VMEM_SHARED((NPAD, cp), jnp.float32),
            pltpu.SemaphoreType.DMA,
            pltpu.SemaphoreType.DMA,
            pltpu.SemaphoreType.DMA,
            pltpu.SemaphoreType.DMA,
        ],
    )
    def k(hp_hbm, asrc_hbm, adst_hbm, srcs_hbm, dsts_hbm, out_hbm,
          src_t, dst_t, asrc_v, adst_v, soff0, soff1, exv0, exv1,
          rb0, rb1, accum, sg0, sg1, ss0, ss1):
        c = lax.axis_index("c")
        s = lax.axis_index("s")
        row0 = s * nrows
        pltpu.sync_copy(srcs_hbm.at[s], src_t)
        pltpu.sync_copy(dsts_hbm.at[s], dst_t)
        soff = (soff0, soff1)
        exv = (exv0, exv1)
        rb = (rb0, rb1)
        sg = (sg0, sg1)
        ss = (ss0, ss1)
        npairs = nch // 2

        for g in range(HEADS // NCORE):
            h = c * (HEADS // NCORE) + g
            pltpu.sync_copy(asrc_hbm.at[h], asrc_v)
            pltpu.sync_copy(adst_hbm.at[h], adst_v)
            hoff = h * NPAD

            def zrow(i, _):
                for f in range(cp // 16):
                    rb0[i, pl.ds(f * 16, 16)] = jnp.zeros((16,), jnp.float32)
                return 0
            lax.fori_loop(0, KE, zrow, 0)
            full = nrows // KE
            for z in range(full):
                pltpu.sync_copy(rb0, accum.at[pl.ds(row0 + z * KE, KE)])
            rem = nrows - full * KE
            if rem:
                pltpu.sync_copy(rb0.at[pl.ds(0, rem)],
                                accum.at[pl.ds(row0 + full * KE, rem)])
            plsc.subcore_barrier()

            def prep(j, b):
                def exstep(v, _):
                    s16 = src_t[j, pl.ds(v * 16, 16)]
                    d16 = dst_t[j, pl.ds(v * 16, 16)]
                    av = plsc.load_gather(asrc_v, [s16])
                    bv = plsc.load_gather(adst_v, [d16])
                    e = av + bv
                    e = jnp.where(e >= 0.0, e, 0.2 * e)
                    exv[b][pl.ds(v * 16, 16)] = jnp.exp(e)
                    soff[b][pl.ds(v * 16, 16)] = s16 + hoff
                    return 0
                lax.fori_loop(0, KE // 16, exstep, 0)

            def gstart(b):
                pltpu.async_copy(hp_hbm.at[soff[b]], rb[b], sg[b])

            def gwait(b):
                pltpu.make_async_copy(hp_hbm.at[soff[b]], rb[b], sg[b]).wait()

            def scale(b):
                def scstep(i, _):
                    m = plsc.load_gather(
                        exv[b], [jnp.full((16,), 1, jnp.int32) * i])
                    for f in range(cp // 16):
                        rb[b][i, pl.ds(f * 16, 16)] = (
                            rb[b][i, pl.ds(f * 16, 16)] * m)
                    return 0
                lax.fori_loop(0, KE, scstep, 0)

            def scstart(j, b):
                pltpu.async_copy(rb[b], accum.at[dst_t.at[j]], ss[b], add=True)

            def scwait(j, b):
                pltpu.make_async_copy(
                    rb[b], accum.at[dst_t.at[j]], ss[b]).wait()

            prep(0, 0)
            gstart(0)

            def pair_body(jp, _):
                j0 = 2 * jp
                prep(j0 + 1, 1)

                @pl.when(jp >= 1)
                def _():
                    scwait(j0 - 1, 1)
                gstart(1)
                gwait(0)
                scale(0)
                scstart(j0, 0)
                @pl.when(jp + 1 < npairs)
                def _():
                    prep(j0 + 2, 0)
                    scwait(j0, 0)
                    gstart(0)
                gwait(1)
                scale(1)
                scstart(j0 + 1, 1)
                return 0

            lax.fori_loop(0, npairs, pair_body, 0)
            scwait(nch - 2, 0)
            scwait(nch - 1, 1)
            plsc.subcore_barrier()

            pltpu.sync_copy(accum.at[pl.ds(row0, nrows)],
                            out_hbm.at[pl.ds(hoff + row0, nrows)])
            plsc.subcore_barrier()

    return k(hp_flat, asrc_p, adst_p, srcs, dsts)


def _gat_sc_layer(h, a, srcs, dsts, out_ch):
    c = out_ch
    asrc_p = jnp.zeros((HEADS, NPAD), jnp.float32).at[:, :N].set(a[:, :8].T)
    adst_p = jnp.zeros((HEADS, NPAD), jnp.float32).at[:, :N].set(a[:, 8:].T)
    ht = jnp.transpose(h.reshape(N, HEADS, c), (1, 0, 2))
    parts = []
    for f0 in range(0, c, 64):
        fw = min(64, c - f0)
        cpp = fw + 16
        hp = jnp.zeros((HEADS, NPAD, cpp), jnp.float32)
        hp = hp.at[:, :N, :fw].set(ht[:, :, f0:f0 + fw])
        hp = hp.at[:, :N, fw].set(1.0)
        agg = _gat_agg_sc(hp.reshape(HEADS * NPAD, cpp), asrc_p, adst_p,
                          srcs, dsts, cpp)
        parts.append((agg.reshape(HEADS, NPAD, cpp), fw))
    return parts


def _attn_mat(p, pre, c):
    aw = jnp.zeros((HEADS * c, 2 * HEADS), jnp.float32)
    for hh in range(HEADS):
        aw = aw.at[hh * c:(hh + 1) * c, hh].set(p[pre + '_asrc'][hh])
        aw = aw.at[hh * c:(hh + 1) * c, HEADS + hh].set(p[pre + '_adst'][hh])
    return aw


def _gatproj_body(x_ref, w_ref, aw_ref, oh_ref, oa_ref):
    h = x_ref[...] @ w_ref[...]
    oh_ref[...] = h
    oa_ref[...] = h @ aw_ref[...]


def _gatproj(x, w, aw, blk=256):
    grid = (pl.cdiv(N, blk),)
    oh, oa = pl.pallas_call(
        _gatproj_body,
        grid=grid,
        in_specs=[pl.BlockSpec((blk, x.shape[1]), lambda i: (i, 0)),
                  pl.BlockSpec(w.shape, lambda i: (0, 0)),
                  pl.BlockSpec(aw.shape, lambda i: (0, 0))],
        out_specs=[pl.BlockSpec((blk, w.shape[1]), lambda i: (i, 0)),
                   pl.BlockSpec((blk, aw.shape[1]), lambda i: (i, 0))],
        out_shape=[jax.ShapeDtypeStruct((N, w.shape[1]), jnp.float32),
                   jax.ShapeDtypeStruct((N, aw.shape[1]), jnp.float32)],
    )(x, w, aw)
    return oh, oa


def _elu(v):
    return jnp.where(v > 0.0, v, jnp.exp(jnp.minimum(v, 0.0)) - 1.0)


def _gat_finish(parts, b_ref):
    cols = []
    for hh in range(HEADS):
        for pr, fw in parts:
            seg = pr[hh]
            cols.append(seg[:, :fw] / (seg[:, fw:fw + 1] + 1e-16))
    v = jnp.concatenate(cols, axis=1) + b_ref[...]
    return _elu(v)


def _gatproj_div_body(nparts, x4_dim, ins_refs):
    part_refs = ins_refs[:nparts]
    b_ref, w_ref, aw_ref, oh_ref, oa_ref = ins_refs[nparts:]
    x4 = _gat_finish([(r[...], r.shape[2] - 16) for r in part_refs], b_ref)
    h = x4 @ w_ref[...]
    oh_ref[...] = h
    oa_ref[...] = h @ aw_ref[...]


def _gatproj_div(parts, b, w, aw, blk=256):
    grid = (pl.cdiv(N, blk),)
    nparts = len(parts)
    part_arrs = [pr for pr, _ in parts]
    in_specs = [pl.BlockSpec((HEADS, blk, pr.shape[2]), lambda i: (0, i, 0))
                for pr in part_arrs]
    in_specs += [pl.BlockSpec((1, b.shape[0]), lambda i: (0, 0)),
                 pl.BlockSpec(w.shape, lambda i: (0, 0)),
                 pl.BlockSpec(aw.shape, lambda i: (0, 0))]

    def body(*refs):
        _gatproj_div_body(nparts, b.shape[0], refs)

    oh, oa = pl.pallas_call(
        body,
        grid=grid,
        in_specs=in_specs,
        out_specs=[pl.BlockSpec((blk, w.shape[1]), lambda i: (i, 0)),
                   pl.BlockSpec((blk, aw.shape[1]), lambda i: (i, 0))],
        out_shape=[jax.ShapeDtypeStruct((N, w.shape[1]), jnp.float32),
                   jax.ShapeDtypeStruct((N, aw.shape[1]), jnp.float32)],
    )(*part_arrs, b.reshape(1, -1), w, aw)
    return oh, oa


def _head_body(x3_ref, p0_ref, b_ref, w1_ref, b1_ref, w2_ref, b2_ref,
               w3_ref, b3_ref, o_ref):
    x6g = _gat_finish([(p0_ref[...], p0_ref.shape[2] - 16)], b_ref)
    x6 = jnp.concatenate([x6g, x3_ref[...]], axis=1)
    h = jnp.maximum(x6 @ w1_ref[...] + b1_ref[...], 0.0)
    h = jnp.maximum(h @ w2_ref[...] + b2_ref[...], 0.0)
    z = h @ w3_ref[...] + b3_ref[...]
    o_ref[...] = 1.0 / (1.0 + jnp.exp(-5.0 * z))


def _head(parts3, gb, x3, p, blk=256):
    pr = parts3[0][0]
    ws = [p['head_w1'], p['head_b1'].reshape(1, -1),
          p['head_w2'], p['head_b2'].reshape(1, -1),
          p['head_w3'], p['head_b3'].reshape(1, -1)]
    grid = (pl.cdiv(N, blk),)
    out = pl.pallas_call(
        _head_body,
        grid=grid,
        in_specs=[pl.BlockSpec((blk, x3.shape[1]), lambda i: (i, 0)),
                  pl.BlockSpec((HEADS, blk, pr.shape[2]), lambda i: (0, i, 0)),
                  pl.BlockSpec((1, gb.shape[0]), lambda i: (0, 0))] +
                 [pl.BlockSpec(w.shape, lambda i: (0, 0)) for w in ws],
        out_specs=pl.BlockSpec((blk, 1), lambda i: (i, 0)),
        out_shape=jax.ShapeDtypeStruct((N, 1), jnp.float32),
    )(x3, pr, gb.reshape(1, -1), *ws)
    return out


def _knn_body(k, nn, hb_ref, hall_ref, o_ref, d_ref):
    hb = hb_ref[...]
    ha = hall_ref[...]
    r = hb.shape[0]
    x2b = jnp.sum(hb * hb, axis=1, keepdims=True)
    x2a = jnp.sum(ha * ha, axis=1)[None, :]
    dot = lax.dot_general(hb, ha, (((1,), (1,)), ((), ())),
                          preferred_element_type=jnp.float32)
    d_ref[...] = 2.0 * dot - x2b - x2a
    iota = lax.broadcasted_iota(jnp.int32, (r, nn), 1)
    kiota = lax.broadcasted_iota(jnp.int32, (r, k), 1)

    def step(kk, idxmat):
        cand = d_ref[...]
        m = jnp.max(cand, axis=1, keepdims=True)
        idxv = jnp.min(jnp.where(cand == m, iota, nn), axis=1, keepdims=True)
        d_ref[...] = jnp.where(iota == idxv, -3.4e38, cand)
        return jnp.where(kiota == kk, idxv, idxmat)

    o_ref[...] = lax.fori_loop(0, k, step, jnp.zeros((r, k), jnp.int32))


def _knn_idx(x, k, blk=512):
    n, c = x.shape
    cpad = (-c) % 128
    if cpad:
        x = jnp.pad(x, ((0, 0), (0, cpad)))
        c += cpad
    npad = (-n) % 512
    nn = n + npad
    if npad:
        x = jnp.pad(x, ((0, npad), (0, 0)), constant_values=1e6)
    grid = (nn // blk,)
    idx = pl.pallas_call(
        functools.partial(_knn_body, k, nn),
        grid=grid,
        in_specs=[pl.BlockSpec((blk, c), lambda i: (i, 0)),
                  pl.BlockSpec((nn, c), lambda i: (0, 0))],
        out_specs=pl.BlockSpec((blk, k), lambda i: (i, 0)),
        out_shape=jax.ShapeDtypeStruct((nn, k), jnp.int32),
        scratch_shapes=[pltpu.VMEM((blk, nn), jnp.float32)],
        compiler_params=pltpu.CompilerParams(
            vmem_limit_bytes=100 * 1024 * 1024),
    )(x, x)
    return idx[:n]


def _pq_body(h_ref, wa_ref, wb_ref, b1_ref, op_ref, oq_ref):
    hb = h_ref[...]
    op_ref[...] = hb @ wa_ref[...] + b1_ref[...]
    oq_ref[...] = hb @ wb_ref[...]


def _rowgather_sc(table, idxp):
    cm = table.shape[1]
    nc = idxp.shape[1]
    npg = nc // 2
    mesh = plsc.VectorSubcoreMesh(core_axis_name="c", subcore_axis_name="s")

    @functools.partial(
        pl.kernel,
        mesh=mesh,
        compiler_params=pltpu.CompilerParams(use_tc_tiling_on_sc=False,
                                             needs_layout_passes=False),
        out_type=jax.ShapeDtypeStruct((NCORE * NSUB * nc * KE, cm),
                                      jnp.float32),
        scratch_types=[
            pltpu.VMEM((nc, KE), jnp.int32),
            pltpu.VMEM((KE, cm), jnp.float32),
            pltpu.VMEM((KE, cm), jnp.float32),
            pltpu.SemaphoreType.DMA,
            pltpu.SemaphoreType.DMA,
        ],
    )
    def k(tab_hbm, idx_hbm, out_hbm, idx_t, b0, b1, s0, s1):
        c = lax.axis_index("c")
        s = lax.axis_index("s")
        w = c * NSUB + s
        pltpu.sync_copy(idx_hbm.at[w], idx_t)
        base = w * nc * KE
        bufs = (b0, b1)
        sems = (s0, s1)

        def gstart(j, b):
            pltpu.async_copy(tab_hbm.at[idx_t.at[j]], bufs[b], sems[b])

        def gwait(j, b):
            pltpu.make_async_copy(
                tab_hbm.at[idx_t.at[j]], bufs[b], sems[b]).wait()

        gstart(0, 0)

        def pair(jp, _):
            j0 = 2 * jp
            gstart(j0 + 1, 1)
            gwait(j0, 0)
            pltpu.sync_copy(b0, out_hbm.at[pl.ds(base + j0 * KE, KE)])

            @pl.when(jp + 1 < npg)
            def _():
                gstart(j0 + 2, 0)
            gwait(j0 + 1, 1)
            pltpu.sync_copy(b1, out_hbm.at[pl.ds(base + (j0 + 1) * KE, KE)])
            return 0

        lax.fori_loop(0, npg, pair, 0)

    return k(table, idxp)


def _econv_body(nk, q_ref, p_ref, w2_ref, b2_ref, o_ref):
    q = q_ref[...]
    pm = p_ref[...]
    blk, cm = pm.shape
    prep = jnp.broadcast_to(pm[:, None, :], (blk, nk, cm)).reshape(blk * nk, cm)
    m = jnp.maximum(q + prep, 0.0)
    z = m @ w2_ref[...] + b2_ref[...]
    o_ref[...] = jnp.max(z.reshape(blk, nk, z.shape[1]), axis=1)


def _edge_conv(h, nk, p, pre, blk=128):
    idx = _knn_idx(h, nk)
    idx = lax.stop_gradient(idx)
    w1 = p[pre + '_w1']
    cin = h.shape[1]
    w1a = w1[:cin] - w1[cin:]
    w1b = w1[cin:]
    cm = w1.shape[1]
    grid_n = pl.cdiv(N, blk)
    pq_grid = (pl.cdiv(N, 256),)
    pmat, qmat = pl.pallas_call(
        _pq_body,
        grid=pq_grid,
        in_specs=[pl.BlockSpec((256, cin), lambda i: (i, 0)),
                  pl.BlockSpec(w1a.shape, lambda i: (0, 0)),
                  pl.BlockSpec(w1b.shape, lambda i: (0, 0)),
                  pl.BlockSpec((1, cm), lambda i: (0, 0))],
        out_specs=[pl.BlockSpec((256, cm), lambda i: (i, 0)),
                   pl.BlockSpec((256, cm), lambda i: (i, 0))],
        out_shape=[jax.ShapeDtypeStruct((N, cm), jnp.float32),
                   jax.ShapeDtypeStruct((N, cm), jnp.float32)],
    )(h, w1a, w1b, p[pre + '_b1'].reshape(1, -1))

    ne = grid_n * blk * nk
    per_w = -(-ne // (NCORE * NSUB * 2 * KE)) * 2 * KE
    ne_sc = NCORE * NSUB * per_w
    idxf = jnp.concatenate(
        [idx.reshape(-1), jnp.zeros((ne_sc - N * nk,), jnp.int32)])
    qg = _rowgather_sc(qmat, idxf.reshape(NCORE * NSUB, -1, KE))

    out = pl.pallas_call(
        functools.partial(_econv_body, nk),
        grid=(grid_n,),
        in_specs=[pl.BlockSpec((blk * nk, cm), lambda i: (i, 0)),
                  pl.BlockSpec((blk, cm), lambda i: (i, 0)),
                  pl.BlockSpec(p[pre + '_w2'].shape, lambda i: (0, 0)),
                  pl.BlockSpec((1, p[pre + '_w2'].shape[1]),
                               lambda i: (0, 0))],
        out_specs=pl.BlockSpec((blk, p[pre + '_w2'].shape[1]),
                               lambda i: (i, 0)),
        out_shape=jax.ShapeDtypeStruct((N, p[pre + '_w2'].shape[1]),
                                       jnp.float32),
    )(qg, pmat, p[pre + '_w2'], p[pre + '_b2'].reshape(1, -1))
    return out


def kernel(x, pos, batch, edge_index, params):
    p = params
    loops = jnp.arange(N, dtype=edge_index.dtype)
    src = jnp.concatenate([edge_index[0], loops])
    dst = jnp.concatenate([edge_index[1], loops])
    e2 = src.shape[0]
    e2p = ((e2 + 2 * NSUB * KE - 1) // (2 * NSUB * KE)) * (2 * NSUB * KE)
    src_p = jnp.concatenate([src, jnp.zeros((e2p - e2,), jnp.int32)])
    dst_p = jnp.concatenate([dst, jnp.full((e2p - e2,), DUMP, jnp.int32)])
    srcs = src_p.reshape(NSUB, -1, KE)
    dsts = dst_p.reshape(NSUB, -1, KE)

    x_surf = x[:, :39]
    xp = _mlp3(x[:, 39:1063], p, 'progen2')
    xd = _mlp3(x[:, 1063:2087], p, 'distarr')
    x0 = jnp.concatenate([x_surf, xp, xd], axis=1)
    x1 = _edge_conv(x0, 20, p, 'conv1')
    x2 = _edge_conv(x1, 20, p, 'conv2')
    x3 = _edge_conv(x2, 20, p, 'conv3')
    x3 = jnp.concatenate([x3, x_surf, xp, xd], axis=1)
    h1, a1 = _gatproj(x3, p['gat1_w'], _attn_mat(p, 'gat1', 128))
    parts1 = _gat_sc_layer(h1, a1, srcs, dsts, 128)
    h2, a2 = _gatproj_div(parts1, p['gat1_b'], p['gat2_w'],
                          _attn_mat(p, 'gat2', 64))
    parts2 = _gat_sc_layer(h2, a2, srcs, dsts, 64)
    h3, a3 = _gatproj_div(parts2, p['gat2_b'], p['gat3_w'],
                          _attn_mat(p, 'gat3', 32))
    parts3 = _gat_sc_layer(h3, a3, srcs, dsts, 32)
    return _head(parts3, p['gat3_b'], x3, p)
